# baseline (speedup 1.0000x reference)
import functools

import jax
import jax.numpy as jnp
from jax import lax
from jax.experimental import pallas as pl
from jax.experimental.pallas import tpu as pltpu
from jax.experimental.pallas import tpu_sc as plsc

TARGET = 98304
NROWS = 65536
NUP = TARGET - NROWS
D = 128
L = 16

NC, NS = 2, 16
NW = NC * NS
UP_PER_W = NUP // NW
CP_PER_W = NROWS // NW
GB = 128
NB = UP_PER_W // GB

_MESH = plsc.VectorSubcoreMesh(
    core_axis_name="c", subcore_axis_name="s", num_cores=NC, num_subcores=NS
)


@functools.partial(
    pl.kernel,
    out_type=jax.ShapeDtypeStruct((TARGET, D), jnp.float32),
    mesh=_MESH,
    scratch_types=[
        pltpu.VMEM((UP_PER_W,), jnp.int32),
        pltpu.VMEM((UP_PER_W,), jnp.int32),
        [pltpu.VMEM((GB, D), jnp.float32) for _ in range(2)],
        [pltpu.VMEM((GB, D), jnp.float32) for _ in range(2)],
        [pltpu.VMEM((GB, D), jnp.float32) for _ in range(2)],
        [pltpu.SemaphoreType.DMA for _ in range(2)],
        [pltpu.SemaphoreType.DMA for _ in range(2)],
        pltpu.SemaphoreType.DMA,
    ],
)
def _hex_unpool(
    x_hbm, idx0_hbm, idx1_hbm, out_hbm, i0v, i1v, r0, r1, ob, sg, so, si
):
    wid = lax.axis_index("s") * NC + lax.axis_index("c")
    ubase = wid * UP_PER_W
    cbase = wid * CP_PER_W

    di0 = pltpu.async_copy(idx0_hbm.at[pl.ds(ubase, UP_PER_W)], i0v, si)
    di1 = pltpu.async_copy(idx1_hbm.at[pl.ds(ubase, UP_PER_W)], i1v, si)

    def copy_phase():
        nchunks = CP_PER_W // GB
        bufs = [r0[0], r0[1], r1[0], r1[1]]
        sems = [sg[0], sg[1], so[0], so[1]]
        loads = [None] * 4
        stores = [None] * 4
        for b in range(4):
            loads[b] = pltpu.async_copy(
                x_hbm.at[pl.ds(cbase + b * GB, GB)], bufs[b], sems[b]
            )
        for c in range(nchunks):
            b = c % 4
            loads[b].wait()
            stores[b] = pltpu.async_copy(
                bufs[b], out_hbm.at[pl.ds(cbase + c * GB, GB)], sems[b]
            )
            if c + 4 < nchunks:
                stores[b].wait()
                loads[b] = pltpu.async_copy(
                    x_hbm.at[pl.ds(cbase + (c + 4) * GB, GB)], bufs[b], sems[b]
                )
                stores[b] = None
        for st in stores:
            if st is not None:
                st.wait()

    def gather_phase():
        di0.wait()
        di1.wait()

        def start_gathers(j, b):
            isl = pl.ds(j * GB, GB)
            d0 = pltpu.async_copy(x_hbm.at[i0v.at[isl]], r0[b], sg[b])
            d1 = pltpu.async_copy(x_hbm.at[i1v.at[isl]], r1[b], sg[b])
            return d0, d1

        pend = start_gathers(0, 0)
        outst = [None, None]
        for j in range(NB):
            b = j % 2
            nxt = None
            if j + 1 < NB:
                nxt = start_gathers(j + 1, 1 - b)
            pend[0].wait()
            pend[1].wait()
            pend = nxt

            if outst[b] is not None:
                outst[b].wait()

            @plsc.parallel_loop(0, GB, 1, unroll=4)
            def avg_body(r, _b=b):
                for c in range(D // L):
                    a = r0[_b][r, pl.ds(c * L, L)]
                    bb = r1[_b][r, pl.ds(c * L, L)]
                    ob[_b][r, pl.ds(c * L, L)] = (a + bb) * 0.5

            outst[b] = pltpu.async_copy(
                ob[b], out_hbm.at[pl.ds(NROWS + ubase + j * GB, GB)], so[b]
            )

        for st in outst:
            if st is not None:
                st.wait()

    even = wid % 2 == 0

    @pl.when(even)
    def _():
        copy_phase()
        gather_phase()

    @pl.when(jnp.logical_not(even))
    def _():
        gather_phase()
        copy_phase()


def kernel(x, upsample_indices):
    idx0 = upsample_indices[:, 0]
    idx1 = upsample_indices[:, 1]
    return _hex_unpool(x, idx0, idx1)


# speedup vs baseline: 1.0677x; 1.0677x over previous
import functools

import jax
import jax.numpy as jnp
from jax import lax
from jax.experimental import pallas as pl
from jax.experimental.pallas import tpu as pltpu
from jax.experimental.pallas import tpu_sc as plsc

TARGET = 98304
NROWS = 65536
NUP = TARGET - NROWS
D = 128
L = 16

NC, NS = 2, 16
NW = NC * NS
UP_PER_W = NUP // NW
CP_PER_W = NROWS // NW
GB = 128
NB = UP_PER_W // GB

_MESH = plsc.VectorSubcoreMesh(
    core_axis_name="c", subcore_axis_name="s", num_cores=NC, num_subcores=NS
)


@functools.partial(
    pl.kernel,
    out_type=jax.ShapeDtypeStruct((TARGET, D), jnp.float32),
    mesh=_MESH,
    scratch_types=[
        pltpu.VMEM((UP_PER_W,), jnp.int32),
        pltpu.VMEM((UP_PER_W,), jnp.int32),
        [pltpu.VMEM((GB, D), jnp.float32) for _ in range(2)],
        [pltpu.VMEM((GB, D), jnp.float32) for _ in range(2)],
        [pltpu.VMEM((GB, D), jnp.float32) for _ in range(2)],
        [pltpu.SemaphoreType.DMA for _ in range(2)],
        [pltpu.SemaphoreType.DMA for _ in range(2)],
        pltpu.SemaphoreType.DMA,
    ],
)
def _hex_unpool(
    x_hbm, idx0_hbm, idx1_hbm, out_hbm, i0v, i1v, r0, r1, ob, sg, so, si
):
    wid = lax.axis_index("s") * NC + lax.axis_index("c")
    ubase = wid * UP_PER_W
    cbase = wid * CP_PER_W

    di0 = pltpu.async_copy(idx0_hbm.at[pl.ds(ubase, UP_PER_W)], i0v, si)
    di1 = pltpu.async_copy(idx1_hbm.at[pl.ds(ubase, UP_PER_W)], i1v, si)

    def copy_phase():
        nchunks = CP_PER_W // GB
        bufs = [r0[0], r0[1], r1[0], r1[1]]
        sems = [sg[0], sg[1], so[0], so[1]]
        loads = [None] * 4
        stores = [None] * 4
        for b in range(4):
            loads[b] = pltpu.async_copy(
                x_hbm.at[pl.ds(cbase + b * GB, GB)], bufs[b], sems[b]
            )
        for c in range(nchunks):
            b = c % 4
            loads[b].wait()
            stores[b] = pltpu.async_copy(
                bufs[b], out_hbm.at[pl.ds(cbase + c * GB, GB)], sems[b]
            )
            if c + 4 < nchunks:
                stores[b].wait()
                loads[b] = pltpu.async_copy(
                    x_hbm.at[pl.ds(cbase + (c + 4) * GB, GB)], bufs[b], sems[b]
                )
                stores[b] = None
        for st in stores:
            if st is not None:
                st.wait()

    def gather_phase():
        di0.wait()
        di1.wait()

        def start_gathers(j, b):
            isl = pl.ds(j * GB, GB)
            d0 = pltpu.async_copy(x_hbm.at[i0v.at[isl]], r0[b], sg[b])
            d1 = pltpu.async_copy(x_hbm.at[i1v.at[isl]], r1[b], sg[b])
            return d0, d1

        pend = start_gathers(0, 0)
        outst = [None, None]
        for j in range(NB):
            b = j % 2
            nxt = None
            if j + 1 < NB:
                nxt = start_gathers(j + 1, 1 - b)
            pend[0].wait()
            pend[1].wait()
            pend = nxt

            if outst[b] is not None:
                outst[b].wait()

            @plsc.parallel_loop(0, GB, 1, unroll=4)
            def avg_body(r, _b=b):
                for c in range(D // L):
                    a = r0[_b][r, pl.ds(c * L, L)]
                    bb = r1[_b][r, pl.ds(c * L, L)]
                    ob[_b][r, pl.ds(c * L, L)] = (a + bb) * 0.5

            outst[b] = pltpu.async_copy(
                ob[b], out_hbm.at[pl.ds(NROWS + ubase + j * GB, GB)], so[b]
            )

        for st in outst:
            if st is not None:
                st.wait()

    copy_phase()
    gather_phase()


def kernel(x, upsample_indices):
    idx0 = upsample_indices[:, 0]
    idx1 = upsample_indices[:, 1]
    return _hex_unpool(x, idx0, idx1)


# speedup vs baseline: 1.0683x; 1.0005x over previous
import functools

import jax
import jax.numpy as jnp
from jax import lax
from jax.experimental import pallas as pl
from jax.experimental.pallas import tpu as pltpu
from jax.experimental.pallas import tpu_sc as plsc

TARGET = 98304
NROWS = 65536
NUP = TARGET - NROWS
D = 128
L = 16

NC, NS = 2, 16
NW = NC * NS
UP_PER_W = NUP // NW
CP_PER_W = NROWS // NW
GB = 128
NB = UP_PER_W // GB
DEPTH = 3

_MESH = plsc.VectorSubcoreMesh(
    core_axis_name="c", subcore_axis_name="s", num_cores=NC, num_subcores=NS
)


@functools.partial(
    pl.kernel,
    out_type=jax.ShapeDtypeStruct((TARGET, D), jnp.float32),
    mesh=_MESH,
    scratch_types=[
        pltpu.VMEM((UP_PER_W,), jnp.int32),
        pltpu.VMEM((UP_PER_W,), jnp.int32),
        [pltpu.VMEM((GB, D), jnp.float32) for _ in range(DEPTH)],
        [pltpu.VMEM((GB, D), jnp.float32) for _ in range(DEPTH)],
        [pltpu.SemaphoreType.DMA for _ in range(DEPTH)],
        [pltpu.SemaphoreType.DMA for _ in range(DEPTH)],
        pltpu.SemaphoreType.DMA,
    ],
)
def _hex_unpool(
    x_hbm, idx0_hbm, idx1_hbm, out_hbm, i0v, i1v, r0, r1, sg, so, si
):
    wid = lax.axis_index("s") * NC + lax.axis_index("c")
    ubase = wid * UP_PER_W
    cbase = wid * CP_PER_W

    di0 = pltpu.async_copy(idx0_hbm.at[pl.ds(ubase, UP_PER_W)], i0v, si)
    di1 = pltpu.async_copy(idx1_hbm.at[pl.ds(ubase, UP_PER_W)], i1v, si)

    nchunks = CP_PER_W // GB
    bufs = r0 + r1
    csems = sg + so
    nbuf = len(bufs)
    loads = [None] * nbuf
    stores = [None] * nbuf
    for b in range(nbuf):
        loads[b] = pltpu.async_copy(x_hbm.at[pl.ds(cbase + b * GB, GB)], bufs[b], csems[b])
    for c in range(nchunks):
        b = c % nbuf
        loads[b].wait()
        stores[b] = pltpu.async_copy(bufs[b], out_hbm.at[pl.ds(cbase + c * GB, GB)], csems[b])
        f = c + 2
        if nbuf <= f < nchunks:
            bf = f % nbuf
            stores[bf].wait()
            loads[bf] = pltpu.async_copy(
                x_hbm.at[pl.ds(cbase + f * GB, GB)], bufs[bf], csems[bf]
            )
            stores[bf] = None
    for st in stores:
        if st is not None:
            st.wait()

    di0.wait()
    di1.wait()

    def start_gathers(j):
        b = j % DEPTH
        isl = pl.ds(j * GB, GB)
        d0 = pltpu.async_copy(x_hbm.at[i0v.at[isl]], r0[b], sg[b])
        d1 = pltpu.async_copy(x_hbm.at[i1v.at[isl]], r1[b], sg[b])
        return d0, d1

    pend = [start_gathers(0), start_gathers(1), None]
    outst = [None] * DEPTH
    for j in range(NB):
        b = j % DEPTH
        if j + 2 < NB:
            bf = (j + 2) % DEPTH
            if outst[bf] is not None:
                outst[bf].wait()
                outst[bf] = None
            pend[bf] = start_gathers(j + 2)
        pend[b][0].wait()
        pend[b][1].wait()

        @plsc.parallel_loop(0, GB, 1, unroll=4)
        def avg_body(r, _b=b):
            for c in range(D // L):
                a = r0[_b][r, pl.ds(c * L, L)]
                bb = r1[_b][r, pl.ds(c * L, L)]
                r0[_b][r, pl.ds(c * L, L)] = (a + bb) * 0.5

        outst[b] = pltpu.async_copy(
            r0[b], out_hbm.at[pl.ds(NROWS + ubase + j * GB, GB)], so[b]
        )

    for st in outst:
        if st is not None:
            st.wait()


def kernel(x, upsample_indices):
    idx0 = upsample_indices[:, 0]
    idx1 = upsample_indices[:, 1]
    return _hex_unpool(x, idx0, idx1)


# speedup vs baseline: 1.1282x; 1.0561x over previous
import functools

import jax
import jax.numpy as jnp
from jax import lax
from jax.experimental import pallas as pl
from jax.experimental.pallas import tpu as pltpu
from jax.experimental.pallas import tpu_sc as plsc

TARGET = 98304
NROWS = 65536
NUP = TARGET - NROWS
D = 128
L = 16

NC, NS = 2, 16
NW = NC * NS
UP_PER_W = NUP // NW
CP_PER_W = NROWS // NW
GB = 128
NB = UP_PER_W // GB
DEPTH = 3

_MESH = plsc.VectorSubcoreMesh(
    core_axis_name="c", subcore_axis_name="s", num_cores=NC, num_subcores=NS
)


@functools.partial(
    pl.kernel,
    out_type=jax.ShapeDtypeStruct((TARGET, D), jnp.float32),
    mesh=_MESH,
    scratch_types=[
        pltpu.VMEM((UP_PER_W,), jnp.int32),
        pltpu.VMEM((UP_PER_W,), jnp.int32),
        [pltpu.VMEM((GB, D), jnp.float32) for _ in range(DEPTH)],
        [pltpu.VMEM((GB, D), jnp.float32) for _ in range(DEPTH)],
        [pltpu.SemaphoreType.DMA for _ in range(DEPTH)],
        [pltpu.SemaphoreType.DMA for _ in range(DEPTH)],
        pltpu.SemaphoreType.DMA,
    ],
)
def _hex_unpool(
    x_hbm, idx0_hbm, idx1_hbm, out_hbm, i0v, i1v, r0, r1, sg, so, si
):
    wid = lax.axis_index("s") * NC + lax.axis_index("c")
    ubase = wid * UP_PER_W
    cbase = wid * CP_PER_W

    di0 = pltpu.async_copy(idx0_hbm.at[pl.ds(ubase, UP_PER_W)], i0v, si)
    di1 = pltpu.async_copy(idx1_hbm.at[pl.ds(ubase, UP_PER_W)], i1v, si)

    nchunks = CP_PER_W // GB
    bufs = r0 + r1
    csems = sg + so
    nbuf = len(bufs)
    loads = [None] * nbuf
    stores = [None] * nbuf
    for b in range(nbuf):
        loads[b] = pltpu.async_copy(x_hbm.at[pl.ds(cbase + b * GB, GB)], bufs[b], csems[b])
    for c in range(nchunks):
        b = c % nbuf
        loads[b].wait()
        stores[b] = pltpu.async_copy(bufs[b], out_hbm.at[pl.ds(cbase + c * GB, GB)], csems[b])
        f = c + 2
        if nbuf <= f < nchunks:
            bf = f % nbuf
            stores[bf].wait()
            loads[bf] = pltpu.async_copy(
                x_hbm.at[pl.ds(cbase + f * GB, GB)], bufs[bf], csems[bf]
            )
            stores[bf] = None
    for st in stores:
        if st is not None:
            st.wait()

    di0.wait()
    di1.wait()

    def start_gathers(j):
        b = j % DEPTH
        isl = pl.ds(j * GB, GB)
        d0 = pltpu.async_copy(x_hbm.at[i0v.at[isl]], r0[b], sg[b])
        d1 = pltpu.async_copy(x_hbm.at[i1v.at[isl]], r1[b], sg[b])
        return d0, d1

    pend = [start_gathers(0), start_gathers(1), None]
    outst = [None] * DEPTH
    for j in range(NB):
        b = j % DEPTH
        if j + 2 < NB:
            bf = (j + 2) % DEPTH
            if outst[bf] is not None:
                outst[bf].wait()
                outst[bf] = None
            pend[bf] = start_gathers(j + 2)
        pend[b][0].wait()
        pend[b][1].wait()

        def avg_body(r, carry, _b=b):
            for c in range(D // L):
                a = r0[_b][r, pl.ds(c * L, L)]
                bb = r1[_b][r, pl.ds(c * L, L)]
                r0[_b][r, pl.ds(c * L, L)] = (a + bb) * 0.5
            return carry

        lax.fori_loop(0, GB, avg_body, 0)

        outst[b] = pltpu.async_copy(
            r0[b], out_hbm.at[pl.ds(NROWS + ubase + j * GB, GB)], so[b]
        )

    for st in outst:
        if st is not None:
            st.wait()


def kernel(x, upsample_indices):
    idx0 = upsample_indices[:, 0]
    idx1 = upsample_indices[:, 1]
    return _hex_unpool(x, idx0, idx1)
